# baseline (speedup 1.0000x reference)
import functools

import jax
import jax.numpy as jnp
from jax import lax
from jax.experimental import pallas as pl
from jax.experimental.pallas import tpu as pltpu
from jax.experimental.pallas import tpu_sc as plsc

_NUM_KNOTS = 30
_N = 33554432
_NC = 2
_NS = 16
_NW = _NC * _NS
_LANES = 16
_CHUNK = 16384
_PER_W = _N // _NW
_N_CHUNKS = _PER_W // _CHUNK
_N_PAIRS = _N_CHUNKS // 2
_TAB = 32


def _sc_spline(x, tab):
    mesh = plsc.VectorSubcoreMesh(
        core_axis_name="c", subcore_axis_name="s",
        num_cores=_NC, num_subcores=_NS)

    @functools.partial(
        pl.kernel,
        out_type=jax.ShapeDtypeStruct((_N,), jnp.float32),
        mesh=mesh,
        scratch_types=[
            pltpu.VMEM((_CHUNK,), jnp.float32),
            pltpu.VMEM((_CHUNK,), jnp.float32),
            pltpu.VMEM((_CHUNK,), jnp.float32),
            pltpu.VMEM((_CHUNK,), jnp.float32),
            pltpu.VMEM((_TAB,), jnp.int32),
            pltpu.SemaphoreType.DMA,
            pltpu.SemaphoreType.DMA,
            pltpu.SemaphoreType.DMA,
            pltpu.SemaphoreType.DMA,
        ],
        compiler_params=pltpu.CompilerParams(needs_layout_passes=False),
    )
    def run(x_hbm, tab_hbm, out_hbm,
            x_v0, x_v1, y_v0, y_v1, a_v,
            sin0, sin1, sout0, sout1):
        wid = lax.axis_index("s") * _NC + lax.axis_index("c")
        pltpu.sync_copy(tab_hbm, a_v)
        base = wid * _PER_W
        x_v = (x_v0, x_v1)
        y_v = (y_v0, y_v1)
        sin = (sin0, sin1)
        sout = (sout0, sout1)

        def in_slice(i):
            return x_hbm.at[pl.ds(base + i * _CHUNK, _CHUNK)]

        def out_slice(i):
            return out_hbm.at[pl.ds(base + i * _CHUNK, _CHUNK)]

        def compute(xb, yb):
            @plsc.parallel_loop(0, _CHUNK, _LANES, unroll=12)
            def vec_body(i):
                xv = xb[pl.ds(i, _LANES)]
                s = xv * jnp.float32(_NUM_KNOTS - 1)
                sc = jnp.minimum(s, jnp.float32(_NUM_KNOTS - 2))
                idx = sc.astype(jnp.int32)
                t = s - idx.astype(jnp.float32)
                w = plsc.load_gather(a_v, [idx])
                c_lo = plsc.bitcast(
                    jnp.bitwise_and(w, jnp.int32(-65536)), jnp.float32)
                d = plsc.bitcast(
                    jnp.left_shift(w, jnp.int32(16)), jnp.float32)
                yb[pl.ds(i, _LANES)] = c_lo + t * d

        pltpu.async_copy(in_slice(0), x_v0, sin0)
        pltpu.async_copy(in_slice(1), x_v1, sin1)

        def pair_body(p, _):
            for b in range(2):
                i = p * 2 + b
                pltpu.make_async_copy(in_slice(i), x_v[b], sin[b]).wait()

                @pl.when(p > 0)
                def _wait_prev_out():
                    pltpu.make_async_copy(y_v[b], out_slice(i), sout[b]).wait()

                compute(x_v[b], y_v[b])
                pltpu.async_copy(y_v[b], out_slice(i), sout[b])

                @pl.when(p < _N_PAIRS - 1)
                def _prefetch_next():
                    pltpu.async_copy(in_slice(i + 2), x_v[b], sin[b])
            return 0

        lax.fori_loop(0, _N_PAIRS, pair_body, 0)

        for b in range(2):
            i = _N_CHUNKS - 2 + b
            pltpu.make_async_copy(y_v[b], out_slice(i), sout[b]).wait()

    return run(x, tab)


def kernel(x, knots, coeffs):
    del knots
    c_lo = coeffs[:_NUM_KNOTS - 1]
    d = coeffs[1:] - coeffs[:_NUM_KNOTS - 1]
    lo16 = jax.lax.bitcast_convert_type(
        c_lo.astype(jnp.bfloat16), jnp.uint16).astype(jnp.uint32)
    d16 = jax.lax.bitcast_convert_type(
        d.astype(jnp.bfloat16), jnp.uint16).astype(jnp.uint32)
    packed = jax.lax.bitcast_convert_type(
        (lo16 << jnp.uint32(16)) | d16, jnp.int32)
    tab = jnp.zeros((_TAB,), jnp.int32).at[:_NUM_KNOTS - 1].set(packed)
    return _sc_spline(x, tab)


# speedup vs baseline: 1.2155x; 1.2155x over previous
import functools

import jax
import jax.numpy as jnp
from jax import lax
from jax.experimental import pallas as pl
from jax.experimental.pallas import tpu as pltpu
from jax.experimental.pallas import tpu_sc as plsc

_NUM_KNOTS = 30
_N = 33554432
_NC = 2
_NS = 16
_NW = _NC * _NS
_LANES = 16
_CHUNK = 16384
_PER_W = _N // _NW
_N_CHUNKS = _PER_W // _CHUNK
_N_PAIRS = _N_CHUNKS // 2
_TAB = 32
_REP = 16


def _sc_spline(x, a_tab, b_tab):
    mesh = plsc.VectorSubcoreMesh(
        core_axis_name="c", subcore_axis_name="s",
        num_cores=_NC, num_subcores=_NS)

    @functools.partial(
        pl.kernel,
        out_type=jax.ShapeDtypeStruct((_N,), jnp.float32),
        mesh=mesh,
        scratch_types=[
            pltpu.VMEM((_CHUNK,), jnp.float32),
            pltpu.VMEM((_CHUNK,), jnp.float32),
            pltpu.VMEM((_CHUNK,), jnp.float32),
            pltpu.VMEM((_CHUNK,), jnp.float32),
            pltpu.VMEM((_TAB * _REP,), jnp.float32),
            pltpu.VMEM((_TAB * _REP,), jnp.float32),
            pltpu.SemaphoreType.DMA,
            pltpu.SemaphoreType.DMA,
            pltpu.SemaphoreType.DMA,
            pltpu.SemaphoreType.DMA,
        ],
        compiler_params=pltpu.CompilerParams(needs_layout_passes=False),
    )
    def run(x_hbm, a_hbm, b_hbm, out_hbm,
            x_v0, x_v1, y_v0, y_v1, a_v, b_v,
            sin0, sin1, sout0, sout1):
        wid = lax.axis_index("s") * _NC + lax.axis_index("c")
        pltpu.sync_copy(a_hbm, a_v)
        pltpu.sync_copy(b_hbm, b_v)
        base = wid * _PER_W
        x_v = (x_v0, x_v1)
        y_v = (y_v0, y_v1)
        sin = (sin0, sin1)
        sout = (sout0, sout1)

        def in_slice(i):
            return x_hbm.at[pl.ds(base + i * _CHUNK, _CHUNK)]

        def out_slice(i):
            return out_hbm.at[pl.ds(base + i * _CHUNK, _CHUNK)]

        lane = lax.iota(jnp.int32, _LANES)

        def compute(xb, yb):
            @plsc.parallel_loop(0, _CHUNK, _LANES, unroll=12)
            def vec_body(i):
                xv = xb[pl.ds(i, _LANES)]
                s = xv * jnp.float32(_NUM_KNOTS - 1)
                sc = jnp.minimum(s, jnp.float32(_NUM_KNOTS - 2))
                idx = sc.astype(jnp.int32)
                vidx = lax.shift_left(idx, 4) + lane
                av = plsc.load_gather(a_v, [vidx])
                bv = plsc.load_gather(b_v, [vidx])
                yb[pl.ds(i, _LANES)] = av + bv * xv

        pltpu.async_copy(in_slice(0), x_v0, sin0)
        pltpu.async_copy(in_slice(1), x_v1, sin1)

        def pair_body(p, _):
            for b in range(2):
                i = p * 2 + b
                pltpu.make_async_copy(in_slice(i), x_v[b], sin[b]).wait()

                @pl.when(p > 0)
                def _wait_prev_out():
                    pltpu.make_async_copy(y_v[b], out_slice(i), sout[b]).wait()

                compute(x_v[b], y_v[b])
                pltpu.async_copy(y_v[b], out_slice(i), sout[b])

                @pl.when(p < _N_PAIRS - 1)
                def _prefetch_next():
                    pltpu.async_copy(in_slice(i + 2), x_v[b], sin[b])
            return 0

        lax.fori_loop(0, _N_PAIRS, pair_body, 0)

        for b in range(2):
            i = _N_CHUNKS - 2 + b
            pltpu.make_async_copy(y_v[b], out_slice(i), sout[b]).wait()

    return run(x, a_tab, b_tab)


def kernel(x, knots, coeffs):
    slope = (coeffs[1:] - coeffs[:-1]) / (knots[1:] - knots[:-1])
    a = coeffs[:-1] - slope * knots[:-1]
    a_tab = jnp.repeat(
        jnp.zeros((_TAB,), jnp.float32).at[:_NUM_KNOTS - 1].set(a), _REP)
    b_tab = jnp.repeat(
        jnp.zeros((_TAB,), jnp.float32).at[:_NUM_KNOTS - 1].set(slope), _REP)
    return _sc_spline(x, a_tab, b_tab)


# speedup vs baseline: 1.2231x; 1.0063x over previous
import functools

import jax
import jax.numpy as jnp
from jax import lax
from jax.experimental import pallas as pl
from jax.experimental.pallas import tpu as pltpu
from jax.experimental.pallas import tpu_sc as plsc

_NUM_KNOTS = 30
_N = 33554432
_NC = 2
_NS = 16
_NW = _NC * _NS
_LANES = 16
_CHUNK = 16384
_PER_W = _N // _NW
_N_CHUNKS = _PER_W // _CHUNK
_N_PAIRS = _N_CHUNKS // 2
_TAB = 32


def _sc_spline(x, a_tab, b_tab):
    mesh = plsc.VectorSubcoreMesh(
        core_axis_name="c", subcore_axis_name="s",
        num_cores=_NC, num_subcores=_NS)

    @functools.partial(
        pl.kernel,
        out_type=jax.ShapeDtypeStruct((_N,), jnp.float32),
        mesh=mesh,
        scratch_types=[
            pltpu.VMEM((_CHUNK,), jnp.float32),
            pltpu.VMEM((_CHUNK,), jnp.float32),
            pltpu.VMEM((_CHUNK,), jnp.float32),
            pltpu.VMEM((_CHUNK,), jnp.float32),
            pltpu.VMEM((_TAB,), jnp.float32),
            pltpu.VMEM((_TAB,), jnp.float32),
            pltpu.SemaphoreType.DMA,
            pltpu.SemaphoreType.DMA,
            pltpu.SemaphoreType.DMA,
            pltpu.SemaphoreType.DMA,
        ],
        compiler_params=pltpu.CompilerParams(needs_layout_passes=False),
    )
    def run(x_hbm, a_hbm, b_hbm, out_hbm,
            x_v0, x_v1, y_v0, y_v1, a_v, b_v,
            sin0, sin1, sout0, sout1):
        wid = lax.axis_index("s") * _NC + lax.axis_index("c")
        pltpu.sync_copy(a_hbm, a_v)
        pltpu.sync_copy(b_hbm, b_v)
        base = wid * _PER_W
        x_v = (x_v0, x_v1)
        y_v = (y_v0, y_v1)
        sin = (sin0, sin1)
        sout = (sout0, sout1)

        def in_slice(i):
            return x_hbm.at[pl.ds(base + i * _CHUNK, _CHUNK)]

        def out_slice(i):
            return out_hbm.at[pl.ds(base + i * _CHUNK, _CHUNK)]

        def compute(xb, yb):
            @plsc.parallel_loop(0, _CHUNK, _LANES, unroll=12)
            def vec_body(i):
                xv = xb[pl.ds(i, _LANES)]
                s = xv * jnp.float32(_NUM_KNOTS - 1)
                sc = jnp.minimum(s, jnp.float32(_NUM_KNOTS - 2))
                idx = sc.astype(jnp.int32)
                av = plsc.load_gather(a_v, [idx])
                bv = plsc.load_gather(b_v, [idx])
                yb[pl.ds(i, _LANES)] = av + bv * xv

        pltpu.async_copy(in_slice(0), x_v0, sin0)
        pltpu.async_copy(in_slice(1), x_v1, sin1)

        def pair_body(p, _):
            for b in range(2):
                i = p * 2 + b
                pltpu.make_async_copy(in_slice(i), x_v[b], sin[b]).wait()

                @pl.when(p > 0)
                def _wait_prev_out():
                    pltpu.make_async_copy(y_v[b], out_slice(i), sout[b]).wait()

                compute(x_v[b], y_v[b])
                pltpu.async_copy(y_v[b], out_slice(i), sout[b])

                @pl.when(p < _N_PAIRS - 1)
                def _prefetch_next():
                    pltpu.async_copy(in_slice(i + 2), x_v[b], sin[b])
            return 0

        lax.fori_loop(0, _N_PAIRS, pair_body, 0)

        for b in range(2):
            i = _N_CHUNKS - 2 + b
            pltpu.make_async_copy(y_v[b], out_slice(i), sout[b]).wait()

    return run(x, a_tab, b_tab)


def kernel(x, knots, coeffs):
    slope = (coeffs[1:] - coeffs[:-1]) / (knots[1:] - knots[:-1])
    a = coeffs[:-1] - slope * knots[:-1]
    a_tab = jnp.zeros((_TAB,), jnp.float32).at[:_NUM_KNOTS - 1].set(a)
    b_tab = jnp.zeros((_TAB,), jnp.float32).at[:_NUM_KNOTS - 1].set(slope)
    return _sc_spline(x, a_tab, b_tab)
